# Initial kernel scaffold; baseline (speedup 1.0000x reference)
import jax
import jax.numpy as jnp
from jax.experimental import pallas as pl


def kernel(input, weights):
    raise NotImplementedError("write your pallas kernel here")


# speedup vs baseline: 3.1457x; 3.1457x over previous
import jax
import jax.numpy as jnp
from jax.experimental import pallas as pl
from jax.experimental.pallas import tpu as pltpu

PAD = 1
SJ = 256


def _body(inpT_ref, w_ref, row1_ref, out_ref):
    w = w_ref[...]
    row1 = row1_ref[...]
    bsz = inpT_ref.shape[1]
    for b in range(bsz):
        mask = inpT_ref[:, b : b + 1] != PAD
        out_ref[b] = jnp.where(mask, w, row1)


def kernel(input, weights):
    bsz, seq_len = input.shape
    d = weights.shape[1]
    inpT = input.T
    wslab = jax.lax.slice(weights, (2, 0), (2 + seq_len, d))
    row1 = jax.lax.slice(weights, (PAD, 0), (PAD + 1, d))
    grid = (seq_len // SJ,)
    return pl.pallas_call(
        _body,
        grid=grid,
        in_specs=[
            pl.BlockSpec((SJ, bsz), lambda j: (j, 0)),
            pl.BlockSpec((SJ, d), lambda j: (j, 0)),
            pl.BlockSpec((1, d), lambda j: (0, 0)),
        ],
        out_specs=pl.BlockSpec((bsz, SJ, d), lambda j: (0, j, 0)),
        out_shape=jax.ShapeDtypeStruct((bsz, seq_len, d), jnp.float32),
    )(inpT, wslab, row1)
